# Initial kernel scaffold; baseline (speedup 1.0000x reference)
import jax
import jax.numpy as jnp
from jax.experimental import pallas as pl


def kernel(x, mask, mask_threshold):
    raise NotImplementedError("write your pallas kernel here")


# speedup vs baseline: 1.0014x; 1.0014x over previous
import functools

import jax
import jax.numpy as jnp
from jax import lax
from jax.experimental import pallas as pl
from jax.experimental.pallas import tpu as pltpu
from jax.experimental.pallas import tpu_sc as plsc

NC = 2
NS = 16
L = 16
NW = NC * NS

N = 32768
D = 1024
TPW = N // NW
K = 64


@functools.partial(
    pl.kernel,
    out_type=jax.ShapeDtypeStruct((N, D), jnp.float32),
    mesh=plsc.VectorSubcoreMesh(core_axis_name="c", subcore_axis_name="s"),
    scratch_types=[
        pltpu.VMEM((TPW,), jnp.int32),
        pltpu.VMEM((TPW,), jnp.float32),
        pltpu.VMEM((TPW,), jnp.float32),
        pltpu.VMEM((K, D), jnp.float32),
        pltpu.SemaphoreType.DMA,
    ],
)
def _select_rows(x_hbm, mask_hbm, out_hbm, idx_v, m0_v, m1_v, rows_v, sem):
    wid = lax.axis_index("s") * NC + lax.axis_index("c")
    base = wid * TPW

    pltpu.sync_copy(mask_hbm.at[0, pl.ds(base, TPW)], m0_v)
    pltpu.sync_copy(mask_hbm.at[1, pl.ds(base, TPW)], m1_v)

    def idx_body(j, carry):
        m0 = m0_v[pl.ds(j * L, L)]
        m1 = m1_v[pl.ds(j * L, L)]
        sel = jnp.where(m0 >= m1, jnp.zeros((L,), jnp.int32),
                        jnp.full((L,), N, jnp.int32))
        ids = base + j * L + lax.iota(jnp.int32, L)
        idx_v[pl.ds(j * L, L)] = ids + sel
        return carry

    lax.fori_loop(0, TPW // L, idx_body, 0)

    def chunk_body(g, carry):
        off = g * K
        pltpu.async_copy(x_hbm.at[idx_v.at[pl.ds(off, K)]], rows_v, sem).wait()
        pltpu.sync_copy(rows_v, out_hbm.at[pl.ds(base + off, K)])
        return carry

    lax.fori_loop(0, TPW // K, chunk_body, 0)


def kernel(x, mask, mask_threshold):
    del mask_threshold
    x_flat = x.reshape(2 * N, D)
    mask_flat = mask.reshape(2, N)
    out = _select_rows(x_flat, mask_flat)
    new_x = out.reshape(4, 8192, D)
    return (new_x, new_x)


# speedup vs baseline: 1.0501x; 1.0486x over previous
import functools

import jax
import jax.numpy as jnp
from jax import lax
from jax.experimental import pallas as pl
from jax.experimental.pallas import tpu as pltpu
from jax.experimental.pallas import tpu_sc as plsc

NC = 2
NS = 16
L = 16
NW = NC * NS

N = 32768
D = 1024
TPW = N // NW
K = 32
G = TPW // K


@functools.partial(
    pl.kernel,
    out_type=jax.ShapeDtypeStruct((N, D), jnp.float32),
    mesh=plsc.VectorSubcoreMesh(core_axis_name="c", subcore_axis_name="s"),
    scratch_types=[
        pltpu.VMEM((TPW,), jnp.int32),
        pltpu.VMEM((TPW,), jnp.float32),
        pltpu.VMEM((TPW,), jnp.float32),
        pltpu.VMEM((2, K, D), jnp.float32),
        pltpu.SemaphoreType.DMA,
        pltpu.SemaphoreType.DMA,
        pltpu.SemaphoreType.DMA,
        pltpu.SemaphoreType.DMA,
    ],
)
def _select_rows(x_hbm, mask_hbm, out_hbm, idx_v, m0_v, m1_v, rows_v,
                 sem_in0, sem_in1, sem_out0, sem_out1):
    wid = lax.axis_index("s") * NC + lax.axis_index("c")
    base = wid * TPW

    pltpu.sync_copy(mask_hbm.at[0, pl.ds(base, TPW)], m0_v)
    pltpu.sync_copy(mask_hbm.at[1, pl.ds(base, TPW)], m1_v)

    def idx_body(j, carry):
        m0 = m0_v[pl.ds(j * L, L)]
        m1 = m1_v[pl.ds(j * L, L)]
        sel = jnp.where(m0 >= m1, jnp.zeros((L,), jnp.int32),
                        jnp.full((L,), N, jnp.int32))
        ids = base + j * L + lax.iota(jnp.int32, L)
        idx_v[pl.ds(j * L, L)] = ids + sel
        return carry

    lax.fori_loop(0, TPW // L, idx_body, 0)

    sem_in = (sem_in0, sem_in1)
    sem_out = (sem_out0, sem_out1)

    def gather_dma(g, s):
        return pltpu.make_async_copy(
            x_hbm.at[idx_v.at[pl.ds(g * K, K)]], rows_v.at[s], sem_in[s])

    def store_dma(g, s):
        return pltpu.make_async_copy(
            rows_v.at[s], out_hbm.at[pl.ds(base + g * K, K)], sem_out[s])

    def chunk_step(g, s):
        so = s ^ 1

        @pl.when(g + 1 < G)
        def _():
            @pl.when(g >= 1)
            def _():
                store_dma(g - 1, so).wait()

            gather_dma(g + 1, so).start()

        gather_dma(g, s).wait()
        store_dma(g, s).start()

    gather_dma(0, 0).start()

    def outer(t, carry):
        chunk_step(2 * t, 0)
        chunk_step(2 * t + 1, 1)
        return carry

    lax.fori_loop(0, G // 2, outer, 0)
    store_dma(G - 2, 0).wait()
    store_dma(G - 1, 1).wait()


def kernel(x, mask, mask_threshold):
    del mask_threshold
    x_flat = x.reshape(2 * N, D)
    mask_flat = mask.reshape(2, N)
    out = _select_rows(x_flat, mask_flat)
    new_x = out.reshape(4, 8192, D)
    return (new_x, new_x)


# speedup vs baseline: 1.2112x; 1.1534x over previous
import functools

import jax
import jax.numpy as jnp
from jax import lax
from jax.experimental import pallas as pl
from jax.experimental.pallas import tpu as pltpu
from jax.experimental.pallas import tpu_sc as plsc

NC = 2
NS = 16
L = 16
NW = NC * NS

N = 32768
D = 1024
TPW = N // NW
K = 32
G = TPW // K


@functools.partial(
    pl.kernel,
    out_type=(
        jax.ShapeDtypeStruct((N, D), jnp.float32),
        jax.ShapeDtypeStruct((N, D), jnp.float32),
    ),
    mesh=plsc.VectorSubcoreMesh(core_axis_name="c", subcore_axis_name="s"),
    scratch_types=[
        pltpu.VMEM((TPW,), jnp.int32),
        pltpu.VMEM((TPW,), jnp.float32),
        pltpu.VMEM((TPW,), jnp.float32),
        pltpu.VMEM((2, K, D), jnp.float32),
        pltpu.SemaphoreType.DMA,
        pltpu.SemaphoreType.DMA,
        pltpu.SemaphoreType.DMA,
        pltpu.SemaphoreType.DMA,
        pltpu.SemaphoreType.DMA,
        pltpu.SemaphoreType.DMA,
    ],
)
def _select_rows(x_hbm, mask_hbm, out1_hbm, out2_hbm, idx_v, m0_v, m1_v,
                 rows_v, sem_in0, sem_in1, sem_a0, sem_a1, sem_b0, sem_b1):
    wid = lax.axis_index("s") * NC + lax.axis_index("c")
    base = wid * TPW

    pltpu.sync_copy(mask_hbm.at[0, pl.ds(base, TPW)], m0_v)
    pltpu.sync_copy(mask_hbm.at[1, pl.ds(base, TPW)], m1_v)

    def idx_body(j, carry):
        m0 = m0_v[pl.ds(j * L, L)]
        m1 = m1_v[pl.ds(j * L, L)]
        sel = jnp.where(m0 >= m1, jnp.zeros((L,), jnp.int32),
                        jnp.full((L,), N, jnp.int32))
        ids = base + j * L + lax.iota(jnp.int32, L)
        idx_v[pl.ds(j * L, L)] = ids + sel
        return carry

    lax.fori_loop(0, TPW // L, idx_body, 0)

    sem_in = (sem_in0, sem_in1)
    sem_a = (sem_a0, sem_a1)
    sem_b = (sem_b0, sem_b1)

    def gather_dma(g, s):
        return pltpu.make_async_copy(
            x_hbm.at[idx_v.at[pl.ds(g * K, K)]], rows_v.at[s], sem_in[s])

    def store_dma(g, s, out_hbm, sem):
        return pltpu.make_async_copy(
            rows_v.at[s], out_hbm.at[pl.ds(base + g * K, K)], sem[s])

    def chunk_step(g, s):
        so = s ^ 1

        @pl.when(g + 1 < G)
        def _():
            @pl.when(g >= 1)
            def _():
                store_dma(g - 1, so, out1_hbm, sem_a).wait()
                store_dma(g - 1, so, out2_hbm, sem_b).wait()

            gather_dma(g + 1, so).start()

        gather_dma(g, s).wait()
        store_dma(g, s, out1_hbm, sem_a).start()
        store_dma(g, s, out2_hbm, sem_b).start()

    gather_dma(0, 0).start()

    def outer(t, carry):
        chunk_step(2 * t, 0)
        chunk_step(2 * t + 1, 1)
        return carry

    lax.fori_loop(0, G // 2, outer, 0)
    store_dma(G - 2, 0, out1_hbm, sem_a).wait()
    store_dma(G - 2, 0, out2_hbm, sem_b).wait()
    store_dma(G - 1, 1, out1_hbm, sem_a).wait()
    store_dma(G - 1, 1, out2_hbm, sem_b).wait()


def kernel(x, mask, mask_threshold):
    del mask_threshold
    x_flat = x.reshape(2 * N, D)
    mask_flat = mask.reshape(2, N)
    o1, o2 = _select_rows(x_flat, mask_flat)
    return (o1.reshape(4, 8192, D), o2.reshape(4, 8192, D))
